# baseline (speedup 1.0000x reference)
import functools

import jax
import jax.numpy as jnp
from jax import lax
from jax.experimental import pallas as pl
from jax.experimental.pallas import tpu as pltpu
from jax.experimental.pallas import tpu_sc as plsc

_NH = 16
_TBLK = 512


def _topk_body(ct_ref, csT_ref, idx_ref, fidx_ref, k0, m2, m3, m4, acc):
    S = csT_ref.shape[1]
    Q = S // 4
    ks = []
    for q in range(4):
        c1 = ct_ref[0, 0:1, :] - csT_ref[0, pl.ds(q * Q, Q), 0:1]
        c2 = ct_ref[0, 1:2, :] - csT_ref[0, pl.ds(q * Q, Q), 1:2]
        d = jnp.sqrt(c1 * c1 + c2 * c2)
        e = lax.bitcast_convert_type(d, jnp.uint32)
        packed = ((e << 2) | q) ^ jnp.uint32(0x80000000)
        ks.append(lax.bitcast_convert_type(packed, jnp.int32))
    def _ce(a, b):
        return jnp.minimum(a, b), jnp.maximum(a, b)
    a, b = _ce(ks[0], ks[1])
    c, d_ = _ce(ks[2], ks[3])
    a, c = _ce(a, c)
    b, d_ = _ce(b, d_)
    b, c = _ce(b, c)
    k0[...], m2[...], m3[...], m4[...] = a, b, c, d_

    iota = lax.broadcasted_iota(jnp.int32, (Q, a.shape[1]), 0)
    umax = jnp.int32(0x7FFFFFFF)

    def body(k, carry):
        kk = k0[...]
        mk = jnp.min(kk, axis=0, keepdims=True)
        prow = jnp.min(jnp.where(kk == mk, iota, Q), axis=0, keepdims=True)
        idx = prow + (mk & 3) * Q
        acc[pl.ds(k, 1), :] = idx
        hit = iota == prow
        m2v, m3v, m4v = m2[...], m3[...], m4[...]
        k0[...] = jnp.where(hit, m2v, kk)
        m2[...] = jnp.where(hit, m3v, m2v)
        m3[...] = jnp.where(hit, m4v, m3v)
        m4[...] = jnp.where(hit, umax, m4v)
        return carry

    lax.fori_loop(0, _NH, body, 0)
    tr = jnp.transpose(acc[...], (1, 0))
    idx_ref[0] = tr
    bi = pl.program_id(0)
    fidx_ref[0] = tr + bi * S


def _topk_call(coords_target, coords_source):
    b, two, t = coords_target.shape
    s = coords_source.shape[2]
    csT = jnp.transpose(coords_source, (0, 2, 1))
    q = s // 4
    return pl.pallas_call(
        _topk_body,
        grid=(b, t // _TBLK),
        in_specs=[
            pl.BlockSpec((1, 2, _TBLK), lambda i, j: (i, 0, j)),
            pl.BlockSpec((1, s, 2), lambda i, j: (i, 0, 0)),
        ],
        out_specs=(pl.BlockSpec((1, _TBLK, _NH), lambda i, j: (i, j, 0)),
                   pl.BlockSpec((1, _TBLK, _NH), lambda i, j: (i, j, 0))),
        out_shape=(jax.ShapeDtypeStruct((b, t, _NH), jnp.int32),
                   jax.ShapeDtypeStruct((b, t, _NH), jnp.int32)),
        scratch_shapes=[pltpu.VMEM((q, _TBLK), jnp.int32) for _ in range(4)]
        + [pltpu.VMEM((_NH, _TBLK), jnp.int32)],
    )(coords_target, csT)


_NWORK = 32
_CHUNK = 128
_TPW = 256


def _sc_gather(xflat, idx3, cflat, ctflat, b, t):
    n_chunks = idx3.shape[1]
    e = xflat.shape[1]
    bs = cflat.shape[0] // 2
    bt = b * t
    nrows = idx3.shape[0] * n_chunks * _CHUNK
    per_w = n_chunks * _CHUNK
    tiles_per_b = t // _TPW
    mesh = plsc.VectorSubcoreMesh(core_axis_name="c", subcore_axis_name="s")

    @functools.partial(
        pl.kernel,
        mesh=mesh,
        compiler_params=pltpu.CompilerParams(
            use_tc_tiling_on_sc=False, needs_layout_passes=False),
        out_type=(
            jax.ShapeDtypeStruct((nrows, e), jnp.float32),
            jax.ShapeDtypeStruct((b, 2, t, _NH), jnp.float32),
        ),
        scratch_types=[
            pltpu.VMEM((n_chunks, _CHUNK), jnp.int32),
            pltpu.VMEM((_CHUNK, e), jnp.float32),
            pltpu.VMEM((_CHUNK, e), jnp.float32),
            pltpu.VMEM(cflat.shape, jnp.float32),
            pltpu.VMEM((2 * _TPW,), jnp.float32),
            pltpu.VMEM((2, _TPW, _NH), jnp.float32),
            pltpu.SemaphoreType.DMA,
            pltpu.SemaphoreType.DMA,
        ],
    )
    def gather_kernel(x_hbm, idx_hbm, ct_hbm, c_hbm, out_hbm, cs_hbm,
                      idxv, rows0, rows1, csv, ctv, cso, sem0, sem1):
        wid = lax.axis_index("s") * 2 + lax.axis_index("c")
        bi = wid // tiles_per_b
        t0 = (wid % tiles_per_b) * _TPW
        pltpu.sync_copy(idx_hbm.at[wid], idxv)
        pltpu.sync_copy(c_hbm, csv)
        for c in range(2):
            pltpu.sync_copy(ct_hbm.at[pl.ds(c * bt + wid * _TPW, _TPW)],
                            ctv.at[pl.ds(c * _TPW, _TPW)])

        def cs_body(ti, carry):
            row = ti // (_CHUNK // _NH)
            col = (ti % (_CHUNK // _NH)) * _NH
            nid = idxv[row, pl.ds(col, _NH)]
            for c in range(2):
                src = plsc.load_gather(csv, [nid + c * bs])
                tgt = plsc.load_gather(
                    ctv, [jnp.full((_NH,), c * _TPW, jnp.int32) + ti])
                cso[c, ti, :] = tgt - src
            return carry

        lax.fori_loop(0, _TPW, cs_body, 0)
        for c in range(2):
            pltpu.sync_copy(cso.at[c], cs_hbm.at[bi, c, pl.ds(t0, _TPW)])

        def body(g, carry):
            j0 = 2 * g
            j1 = 2 * g + 1
            cp0 = pltpu.async_copy(x_hbm.at[idxv.at[j0]], rows0, sem0)
            cp1 = pltpu.async_copy(x_hbm.at[idxv.at[j1]], rows1, sem1)
            cp0.wait()
            pltpu.sync_copy(rows0,
                            out_hbm.at[pl.ds(wid * per_w + j0 * _CHUNK, _CHUNK)])
            cp1.wait()
            pltpu.sync_copy(rows1,
                            out_hbm.at[pl.ds(wid * per_w + j1 * _CHUNK, _CHUNK)])
            return carry

        lax.fori_loop(0, n_chunks // 2, body, 0)

    return gather_kernel(xflat, idx3, ctflat, cflat)


def kernel(x, coords_target, coords_source):
    b, s, e = x.shape
    t = coords_target.shape[2]
    idx, flat_idx = _topk_call(coords_target, coords_source)
    idx3 = flat_idx.reshape(_NWORK, (b * t * _NH) // (_NWORK * _CHUNK), _CHUNK)
    cflat = jnp.transpose(coords_source, (1, 0, 2)).reshape(2 * b * s)
    ctflat = jnp.transpose(coords_target, (1, 0, 2)).reshape(2 * b * t)
    rows, cs = _sc_gather(x.reshape(b * s, e), idx3, cflat, ctflat, b, t)
    x_bs = rows.reshape(b, t, _NH, e)
    return (x_bs, idx, cs)


# speedup vs baseline: 1.1467x; 1.1467x over previous
import functools

import jax
import jax.numpy as jnp
from jax import lax
from jax.experimental import pallas as pl
from jax.experimental.pallas import tpu as pltpu
from jax.experimental.pallas import tpu_sc as plsc

_NH = 16
_TBLK = 512


def _topk_body(ct_ref, csT_ref, idx_ref, fidx_ref, k0, m2, m3, m4, acc):
    S = csT_ref.shape[1]
    Q = S // 4
    ks = []
    for q in range(4):
        c1 = ct_ref[0, 0:1, :] - csT_ref[0, pl.ds(q * Q, Q), 0:1]
        c2 = ct_ref[0, 1:2, :] - csT_ref[0, pl.ds(q * Q, Q), 1:2]
        d = jnp.sqrt(c1 * c1 + c2 * c2)
        e = lax.bitcast_convert_type(d, jnp.uint32)
        packed = ((e << 2) | q) ^ jnp.uint32(0x80000000)
        ks.append(lax.bitcast_convert_type(packed, jnp.int32))
    def _ce(a, b):
        return jnp.minimum(a, b), jnp.maximum(a, b)
    a, b = _ce(ks[0], ks[1])
    c, d_ = _ce(ks[2], ks[3])
    a, c = _ce(a, c)
    b, d_ = _ce(b, d_)
    b, c = _ce(b, c)
    k0[...], m2[...], m3[...], m4[...] = a, b, c, d_

    iota = lax.broadcasted_iota(jnp.int32, (Q, a.shape[1]), 0)
    umax = jnp.int32(0x7FFFFFFF)

    def body(k, carry):
        kk = k0[...]
        mk = jnp.min(kk, axis=0, keepdims=True)
        prow = jnp.min(jnp.where(kk == mk, iota, Q), axis=0, keepdims=True)
        idx = prow + (mk & 3) * Q
        acc[pl.ds(k, 1), :] = idx
        hit = iota == prow
        nxt = jnp.where(kk < m2[...], m2[...],
                        jnp.where(kk < m3[...], m3[...],
                                  jnp.where(kk < m4[...], m4[...], umax)))
        k0[...] = jnp.where(hit, nxt, kk)
        return carry

    lax.fori_loop(0, _NH, body, 0)
    tr = jnp.transpose(acc[...], (1, 0))
    idx_ref[0] = tr
    bi = pl.program_id(0)
    fidx_ref[0] = tr + bi * S


def _topk_call(coords_target, coords_source):
    b, two, t = coords_target.shape
    s = coords_source.shape[2]
    csT = jnp.transpose(coords_source, (0, 2, 1))
    q = s // 4
    return pl.pallas_call(
        _topk_body,
        grid=(b, t // _TBLK),
        in_specs=[
            pl.BlockSpec((1, 2, _TBLK), lambda i, j: (i, 0, j)),
            pl.BlockSpec((1, s, 2), lambda i, j: (i, 0, 0)),
        ],
        out_specs=(pl.BlockSpec((1, _TBLK, _NH), lambda i, j: (i, j, 0)),
                   pl.BlockSpec((1, _TBLK, _NH), lambda i, j: (i, j, 0))),
        out_shape=(jax.ShapeDtypeStruct((b, t, _NH), jnp.int32),
                   jax.ShapeDtypeStruct((b, t, _NH), jnp.int32)),
        scratch_shapes=[pltpu.VMEM((q, _TBLK), jnp.int32) for _ in range(4)]
        + [pltpu.VMEM((_NH, _TBLK), jnp.int32)],
    )(coords_target, csT)


_NWORK = 32
_CHUNK = 128
_TPW = 256


def _sc_gather(xflat, idx3, cflat, ctflat, b, t):
    n_chunks = idx3.shape[1]
    e = xflat.shape[1]
    bs = cflat.shape[0] // 2
    bt = b * t
    nrows = idx3.shape[0] * n_chunks * _CHUNK
    per_w = n_chunks * _CHUNK
    tiles_per_b = t // _TPW
    mesh = plsc.VectorSubcoreMesh(core_axis_name="c", subcore_axis_name="s")

    @functools.partial(
        pl.kernel,
        mesh=mesh,
        compiler_params=pltpu.CompilerParams(
            use_tc_tiling_on_sc=False, needs_layout_passes=False),
        out_type=(
            jax.ShapeDtypeStruct((nrows, e), jnp.float32),
            jax.ShapeDtypeStruct((b, 2, t, _NH), jnp.float32),
        ),
        scratch_types=[
            pltpu.VMEM((n_chunks, _CHUNK), jnp.int32),
            pltpu.VMEM((_CHUNK, e), jnp.float32),
            pltpu.VMEM((_CHUNK, e), jnp.float32),
            pltpu.VMEM(cflat.shape, jnp.float32),
            pltpu.VMEM((2 * _TPW,), jnp.float32),
            pltpu.VMEM((2, _TPW, _NH), jnp.float32),
            pltpu.SemaphoreType.DMA,
            pltpu.SemaphoreType.DMA,
        ],
    )
    def gather_kernel(x_hbm, idx_hbm, ct_hbm, c_hbm, out_hbm, cs_hbm,
                      idxv, rows0, rows1, csv, ctv, cso, sem0, sem1):
        wid = lax.axis_index("s") * 2 + lax.axis_index("c")
        bi = wid // tiles_per_b
        t0 = (wid % tiles_per_b) * _TPW
        pltpu.sync_copy(idx_hbm.at[wid], idxv)
        pltpu.sync_copy(c_hbm, csv)
        for c in range(2):
            pltpu.sync_copy(ct_hbm.at[pl.ds(c * bt + wid * _TPW, _TPW)],
                            ctv.at[pl.ds(c * _TPW, _TPW)])

        def cs_body(ti, carry):
            row = ti // (_CHUNK // _NH)
            col = (ti % (_CHUNK // _NH)) * _NH
            nid = idxv[row, pl.ds(col, _NH)]
            for c in range(2):
                src = plsc.load_gather(csv, [nid + c * bs])
                tgt = plsc.load_gather(
                    ctv, [jnp.full((_NH,), c * _TPW, jnp.int32) + ti])
                cso[c, ti, :] = tgt - src
            return carry

        lax.fori_loop(0, _TPW, cs_body, 0)
        for c in range(2):
            pltpu.sync_copy(cso.at[c], cs_hbm.at[bi, c, pl.ds(t0, _TPW)])

        def body(g, carry):
            j0 = 2 * g
            j1 = 2 * g + 1
            cp0 = pltpu.async_copy(x_hbm.at[idxv.at[j0]], rows0, sem0)
            cp1 = pltpu.async_copy(x_hbm.at[idxv.at[j1]], rows1, sem1)
            cp0.wait()
            pltpu.sync_copy(rows0,
                            out_hbm.at[pl.ds(wid * per_w + j0 * _CHUNK, _CHUNK)])
            cp1.wait()
            pltpu.sync_copy(rows1,
                            out_hbm.at[pl.ds(wid * per_w + j1 * _CHUNK, _CHUNK)])
            return carry

        lax.fori_loop(0, n_chunks // 2, body, 0)

    return gather_kernel(xflat, idx3, ctflat, cflat)


def kernel(x, coords_target, coords_source):
    b, s, e = x.shape
    t = coords_target.shape[2]
    idx, flat_idx = _topk_call(coords_target, coords_source)
    idx3 = flat_idx.reshape(_NWORK, (b * t * _NH) // (_NWORK * _CHUNK), _CHUNK)
    cflat = jnp.transpose(coords_source, (1, 0, 2)).reshape(2 * b * s)
    ctflat = jnp.transpose(coords_target, (1, 0, 2)).reshape(2 * b * t)
    rows, cs = _sc_gather(x.reshape(b * s, e), idx3, cflat, ctflat, b, t)
    x_bs = rows.reshape(b, t, _NH, e)
    return (x_bs, idx, cs)


# speedup vs baseline: 1.1626x; 1.0138x over previous
import functools

import jax
import jax.numpy as jnp
from jax import lax
from jax.experimental import pallas as pl
from jax.experimental.pallas import tpu as pltpu
from jax.experimental.pallas import tpu_sc as plsc

_NH = 16
_TBLK = 512


def _topk_body(ct_ref, csT_ref, idx_ref, fidx_ref, k0, m2, m3, m4, acc):
    S = csT_ref.shape[1]
    Q = S // 4
    ks = []
    for q in range(4):
        c1 = ct_ref[0, 0:1, :] - csT_ref[0, pl.ds(q * Q, Q), 0:1]
        c2 = ct_ref[0, 1:2, :] - csT_ref[0, pl.ds(q * Q, Q), 1:2]
        d = jnp.sqrt(c1 * c1 + c2 * c2)
        e = lax.bitcast_convert_type(d, jnp.uint32)
        packed = ((e << 2) | q) ^ jnp.uint32(0x80000000)
        ks.append(lax.bitcast_convert_type(packed, jnp.int32))
    def _ce(a, b):
        return jnp.minimum(a, b), jnp.maximum(a, b)
    a, b = _ce(ks[0], ks[1])
    c, d_ = _ce(ks[2], ks[3])
    a, c = _ce(a, c)
    b, d_ = _ce(b, d_)
    b, c = _ce(b, c)
    k0[...], m2[...], m3[...], m4[...] = a, b, c, d_

    iota = lax.broadcasted_iota(jnp.int32, (Q, a.shape[1]), 0)
    umax = jnp.int32(0x7FFFFFFF)

    def body(k, carry):
        kk = k0[...]
        mk = jnp.min(kk, axis=0, keepdims=True)
        prow = jnp.min(jnp.where(kk == mk, iota, Q), axis=0, keepdims=True)
        idx = prow + (mk & 3) * Q
        acc[pl.ds(k, 1), :] = idx
        hit = iota == prow
        nxt = jnp.where(kk < m2[...], m2[...],
                        jnp.where(kk < m3[...], m3[...],
                                  jnp.where(kk < m4[...], m4[...], umax)))
        k0[...] = jnp.where(hit, nxt, kk)
        return carry

    lax.fori_loop(0, _NH, body, 0)
    tr = jnp.transpose(acc[...], (1, 0))
    idx_ref[0] = tr
    bi = pl.program_id(0)
    fidx_ref[0] = tr + bi * S


def _topk_call(coords_target, coords_source):
    b, two, t = coords_target.shape
    s = coords_source.shape[2]
    csT = jnp.transpose(coords_source, (0, 2, 1))
    q = s // 4
    return pl.pallas_call(
        _topk_body,
        grid=(b, t // _TBLK),
        in_specs=[
            pl.BlockSpec((1, 2, _TBLK), lambda i, j: (i, 0, j)),
            pl.BlockSpec((1, s, 2), lambda i, j: (i, 0, 0)),
        ],
        out_specs=(pl.BlockSpec((1, _TBLK, _NH), lambda i, j: (i, j, 0)),
                   pl.BlockSpec((1, _TBLK, _NH), lambda i, j: (i, j, 0))),
        out_shape=(jax.ShapeDtypeStruct((b, t, _NH), jnp.int32),
                   jax.ShapeDtypeStruct((b, t, _NH), jnp.int32)),
        scratch_shapes=[pltpu.VMEM((q, _TBLK), jnp.int32) for _ in range(4)]
        + [pltpu.VMEM((_NH, _TBLK), jnp.int32)],
    )(coords_target, csT)


_NWORK = 32
_CHUNK = 128
_TPW = 256


def _sc_gather(xflat, idx3, cflat, ctflat, b, t):
    n_chunks = idx3.shape[1]
    e = xflat.shape[1]
    bs = cflat.shape[0] // 2
    bt = b * t
    nrows = idx3.shape[0] * n_chunks * _CHUNK
    per_w = n_chunks * _CHUNK
    tiles_per_b = t // _TPW
    mesh = plsc.VectorSubcoreMesh(core_axis_name="c", subcore_axis_name="s")

    @functools.partial(
        pl.kernel,
        mesh=mesh,
        compiler_params=pltpu.CompilerParams(
            use_tc_tiling_on_sc=False, needs_layout_passes=False),
        out_type=(
            jax.ShapeDtypeStruct((nrows, e), jnp.float32),
            jax.ShapeDtypeStruct((b, 2, t, _NH), jnp.float32),
        ),
        scratch_types=[
            pltpu.VMEM((n_chunks, _CHUNK), jnp.int32),
            pltpu.VMEM((_CHUNK, e), jnp.float32),
            pltpu.VMEM((_CHUNK, e), jnp.float32),
            pltpu.VMEM((_CHUNK, e), jnp.float32),
            pltpu.VMEM((_CHUNK, e), jnp.float32),
            pltpu.VMEM(cflat.shape, jnp.float32),
            pltpu.VMEM((2 * _TPW,), jnp.float32),
            pltpu.VMEM((2, _TPW, _NH), jnp.float32),
            pltpu.SemaphoreType.DMA,
            pltpu.SemaphoreType.DMA,
            pltpu.SemaphoreType.DMA,
            pltpu.SemaphoreType.DMA,
        ],
    )
    def gather_kernel(x_hbm, idx_hbm, ct_hbm, c_hbm, out_hbm, cs_hbm,
                      idxv, rows0, rows1, rows2, rows3, csv, ctv, cso,
                      sem0, sem1, sem2, sem3):
        wid = lax.axis_index("s") * 2 + lax.axis_index("c")
        bi = wid // tiles_per_b
        t0 = (wid % tiles_per_b) * _TPW
        pltpu.sync_copy(idx_hbm.at[wid], idxv)
        pltpu.sync_copy(c_hbm, csv)
        for c in range(2):
            pltpu.sync_copy(ct_hbm.at[pl.ds(c * bt + wid * _TPW, _TPW)],
                            ctv.at[pl.ds(c * _TPW, _TPW)])

        def cs_body(ti, carry):
            row = ti // (_CHUNK // _NH)
            col = (ti % (_CHUNK // _NH)) * _NH
            nid = idxv[row, pl.ds(col, _NH)]
            for c in range(2):
                src = plsc.load_gather(csv, [nid + c * bs])
                tgt = plsc.load_gather(
                    ctv, [jnp.full((_NH,), c * _TPW, jnp.int32) + ti])
                cso[c, ti, :] = tgt - src
            return carry

        lax.fori_loop(0, _TPW, cs_body, 0)
        for c in range(2):
            pltpu.sync_copy(cso.at[c], cs_hbm.at[bi, c, pl.ds(t0, _TPW)])

        bufs = (rows0, rows1, rows2, rows3)
        sems = (sem0, sem1, sem2, sem3)

        def body(g, carry):
            cps = []
            for i in range(4):
                cps.append(pltpu.async_copy(
                    x_hbm.at[idxv.at[4 * g + i]], bufs[i], sems[i]))
            for i in range(4):
                cps[i].wait()
                pltpu.sync_copy(
                    bufs[i],
                    out_hbm.at[pl.ds(wid * per_w + (4 * g + i) * _CHUNK,
                                     _CHUNK)])
            return carry

        lax.fori_loop(0, n_chunks // 4, body, 0)

    return gather_kernel(xflat, idx3, ctflat, cflat)


def kernel(x, coords_target, coords_source):
    b, s, e = x.shape
    t = coords_target.shape[2]
    idx, flat_idx = _topk_call(coords_target, coords_source)
    idx3 = flat_idx.reshape(_NWORK, (b * t * _NH) // (_NWORK * _CHUNK), _CHUNK)
    cflat = jnp.transpose(coords_source, (1, 0, 2)).reshape(2 * b * s)
    ctflat = jnp.transpose(coords_target, (1, 0, 2)).reshape(2 * b * t)
    rows, cs = _sc_gather(x.reshape(b * s, e), idx3, cflat, ctflat, b, t)
    x_bs = rows.reshape(b, t, _NH, e)
    return (x_bs, idx, cs)
